# speedup vs baseline: 1.3434x; 1.3434x over previous
import functools

import jax
import jax.numpy as jnp
from jax import lax
from jax.experimental import pallas as pl
from jax.experimental.pallas import tpu as pltpu
from jax.experimental.pallas import tpu_sc as plsc

N = 16384
D = 256
K = 1024

BN = 1024
GRID = N // BN

_SC_INFO = plsc.get_sparse_core_info()
_NC = _SC_INFO.num_cores
_NS = _SC_INFO.num_subcores
NW = _NC * _NS
B_PER_W = N // NW
CHUNK = 256


def _tc_body(x_ref, e_ref, x2_ref, e2_ref, idx_ref, w_ref):
    i = pl.program_id(0)
    x = x_ref[...]
    e = e_ref[...]
    s = jnp.dot(x, e, preferred_element_type=jnp.float32)
    d = (x2_ref[...] - 2.0 * s) + e2_ref[...]
    m = jnp.min(d, axis=1, keepdims=True)
    ii = lax.broadcasted_iota(jnp.int32, (BN, K), 1)
    idx_ref[...] = jnp.min(jnp.where(d == m, ii, K), axis=1)

    @pl.when(i == 0)
    def _():
        w_ref[...] = e.T


def _encode(x, embeddings, x2, e2):
    return pl.pallas_call(
        _tc_body,
        grid=(GRID,),
        in_specs=[
            pl.BlockSpec((BN, D), lambda i: (i, 0)),
            pl.BlockSpec((D, K), lambda i: (0, 0)),
            pl.BlockSpec((BN, 1), lambda i: (i, 0)),
            pl.BlockSpec((1, K), lambda i: (0, 0)),
        ],
        out_specs=[
            pl.BlockSpec((BN,), lambda i: (i,)),
            pl.BlockSpec((K, D), lambda i: (0, 0)),
        ],
        out_shape=[
            jax.ShapeDtypeStruct((N,), jnp.int32),
            jax.ShapeDtypeStruct((K, D), jnp.float32),
        ],
        compiler_params=pltpu.CompilerParams(
            dimension_semantics=("arbitrary",),
        ),
    )(x, embeddings, x2, e2)


@functools.partial(
    pl.kernel,
    out_type=jax.ShapeDtypeStruct((N, D), jnp.float32),
    mesh=plsc.VectorSubcoreMesh(core_axis_name="c", subcore_axis_name="s"),
    scratch_types=[
        pltpu.VMEM((CHUNK,), jnp.int32),
        pltpu.VMEM((CHUNK, D), jnp.float32),
        pltpu.SemaphoreType.DMA,
    ],
)
def _gather(w_hbm, idx_hbm, out_hbm, idx_v, rows_v, sem):
    wid = lax.axis_index("s") * _NC + lax.axis_index("c")
    for c in range(B_PER_W // CHUNK):
        base = wid * B_PER_W + c * CHUNK
        pltpu.sync_copy(idx_hbm.at[pl.ds(base, CHUNK)], idx_v)
        pltpu.async_copy(w_hbm.at[idx_v], rows_v, sem).wait()
        pltpu.sync_copy(rows_v, out_hbm.at[pl.ds(base, CHUNK)])


def kernel(x, embeddings):
    x2 = jnp.sum(x ** 2, axis=1, keepdims=True)
    e2 = jnp.sum(embeddings ** 2, axis=0, keepdims=True)
    idx, w = _encode(x, embeddings, x2, e2)
    return _gather(w, idx)


# speedup vs baseline: 1.5109x; 1.1247x over previous
import functools

import jax
import jax.numpy as jnp
from jax import lax
from jax.experimental import pallas as pl
from jax.experimental.pallas import tpu as pltpu
from jax.experimental.pallas import tpu_sc as plsc

N = 16384
D = 256
K = 1024

BN = 1024
GRID = N // BN

_SC_INFO = plsc.get_sparse_core_info()
_NC = _SC_INFO.num_cores
_NS = _SC_INFO.num_subcores
NW = _NC * _NS
B_PER_W = N // NW
CHUNK = 256


def _tc_body(x_ref, e_ref, x2_ref, e2_ref, idx_ref, w_ref):
    i = pl.program_id(0)
    x = x_ref[...]
    e = e_ref[...]
    s = jnp.dot(x, e, preferred_element_type=jnp.float32)
    d = (x2_ref[...] - 2.0 * s) + e2_ref[...]
    idx_ref[...] = jnp.argmin(d, axis=1).astype(jnp.int32)

    @pl.when(i == 0)
    def _():
        w_ref[...] = e.T


def _encode(x, embeddings, x2, e2):
    return pl.pallas_call(
        _tc_body,
        grid=(GRID,),
        in_specs=[
            pl.BlockSpec((BN, D), lambda i: (i, 0)),
            pl.BlockSpec((D, K), lambda i: (0, 0)),
            pl.BlockSpec((BN, 1), lambda i: (i, 0)),
            pl.BlockSpec((1, K), lambda i: (0, 0)),
        ],
        out_specs=[
            pl.BlockSpec((BN,), lambda i: (i,)),
            pl.BlockSpec((K, D), lambda i: (0, 0)),
        ],
        out_shape=[
            jax.ShapeDtypeStruct((N,), jnp.int32),
            jax.ShapeDtypeStruct((K, D), jnp.float32),
        ],
        compiler_params=pltpu.CompilerParams(
            dimension_semantics=("arbitrary",),
        ),
    )(x, embeddings, x2, e2)


@functools.partial(
    pl.kernel,
    out_type=jax.ShapeDtypeStruct((N, D), jnp.float32),
    mesh=plsc.VectorSubcoreMesh(core_axis_name="c", subcore_axis_name="s"),
    scratch_types=[
        pltpu.VMEM((CHUNK,), jnp.int32),
        pltpu.VMEM((CHUNK, D), jnp.float32),
        pltpu.SemaphoreType.DMA,
    ],
)
def _gather(w_hbm, idx_hbm, out_hbm, idx_v, rows_v, sem):
    wid = lax.axis_index("s") * _NC + lax.axis_index("c")
    for c in range(B_PER_W // CHUNK):
        base = wid * B_PER_W + c * CHUNK
        pltpu.sync_copy(idx_hbm.at[pl.ds(base, CHUNK)], idx_v)
        pltpu.async_copy(w_hbm.at[idx_v], rows_v, sem).wait()
        pltpu.sync_copy(rows_v, out_hbm.at[pl.ds(base, CHUNK)])


def kernel(x, embeddings):
    x2 = jnp.sum(x ** 2, axis=1, keepdims=True)
    e2 = jnp.sum(embeddings ** 2, axis=0, keepdims=True)
    idx, w = _encode(x, embeddings, x2, e2)
    return _gather(w, idx)
